# baseline (speedup 1.0000x reference)
import functools

import jax
import jax.numpy as jnp
from jax import lax
from jax.experimental import pallas as pl
from jax.experimental.pallas import tpu as pltpu
from jax.experimental.pallas import tpu_sc as plsc

B = 200
LANES = 16
BP = 208
NV = BP // LANES
NC, NS = 2, 16
NW = NC * NS


def _sc_pool(n_rows, cin, n_out_pad, E, K):
    row_w = cin * BP
    RPT = n_out_pad // NS
    Z = min(16, RPT)
    nz = RPT // Z
    mesh = plsc.VectorSubcoreMesh(core_axis_name="c", subcore_axis_name="s")

    @functools.partial(
        pl.kernel,
        mesh=mesh,
        compiler_params=pltpu.CompilerParams(use_tc_tiling_on_sc=False),
        out_type=jax.ShapeDtypeStruct((NC, n_out_pad, BP), jnp.float32),
        scratch_types=[
            pltpu.VMEM((E,), jnp.int32),
            pltpu.VMEM((E,), jnp.int32),
            pltpu.VMEM((cin, E), jnp.float32),
            pltpu.VMEM((E, row_w), jnp.float32),
            pltpu.VMEM((E, BP), jnp.float32),
            pltpu.VMEM((Z, BP), jnp.float32),
            pltpu.VMEM_SHARED((n_out_pad, BP), jnp.float32),
            pltpu.SemaphoreType.DMA,
        ],
    )
    def kfn(h_hbm, src_hbm, dst_hbm, w_hbm, out_hbm,
            src_v, dst_v, w_v, rows_v, out_v, zbuf, acc, sem):
        cid = lax.axis_index("c")
        sid = lax.axis_index("s")
        wid = cid * NS + sid

        zv = jnp.zeros((LANES,), jnp.float32)
        for i in range(Z):
            for j in range(NV):
                zbuf[i, pl.ds(j * LANES, LANES)] = zv
        for t in range(nz):
            pltpu.sync_copy(zbuf, acc.at[pl.ds(sid * RPT + t * Z, Z)])
        plsc.subcore_barrier()

        def chunk_body(k, carry):
            base = pl.multiple_of((wid * K + k) * E, 8)
            pltpu.sync_copy(src_hbm.at[pl.ds(base, E)], src_v)
            pltpu.sync_copy(dst_hbm.at[pl.ds(base, E)], dst_v)
            for c in range(cin):
                pltpu.sync_copy(w_hbm.at[c, pl.ds(base, E)], w_v.at[c])
            pltpu.async_copy(h_hbm.at[src_v], rows_v, sem).wait()

            def group_body(gi, c2):
                wv = [w_v[c, pl.ds(gi * LANES, LANES)] for c in range(cin)]
                for l in range(LANES):
                    e = gi * LANES + l
                    ws = [wv[c][l] for c in range(cin)]
                    for j in range(NV):
                        sl = pl.ds(j * LANES, LANES)
                        r = rows_v[e, sl] * ws[0]
                        for c in range(1, cin):
                            r += rows_v[e, pl.ds(c * BP + j * LANES, LANES)] * ws[c]
                        out_v[e, sl] = r
                return c2

            lax.fori_loop(0, E // LANES, group_body, 0)
            pltpu.sync_copy(out_v, acc.at[dst_v], add=True)
            return carry

        lax.fori_loop(0, K, chunk_body, 0)
        plsc.subcore_barrier()
        pltpu.sync_copy(acc.at[pl.ds(sid * RPT, RPT)],
                        out_hbm.at[cid, pl.ds(sid * RPT, RPT)])

    return kfn


def _bn_relu(part, g, be):
    np_rows = part.shape[1]

    def body(p_ref, g_ref, be_ref, o_ref):
        t = p_ref[0] + p_ref[1]
        lane = lax.broadcasted_iota(jnp.int32, (np_rows, BP), 1)
        mask = lane < B
        m = jnp.sum(t, axis=1, keepdims=True) * (1.0 / B)
        d = jnp.where(mask, t - m, 0.0)
        v = jnp.sum(d * d, axis=1, keepdims=True) * (1.0 / B)
        y = d * lax.rsqrt(v + 1e-5) * g_ref[...] + be_ref[...]
        o_ref[...] = jnp.where(mask, jnp.maximum(y, 0.0), 0.0)

    return pl.pallas_call(
        body,
        out_shape=jax.ShapeDtypeStruct((np_rows, BP), jnp.float32),
    )(part, g.reshape(np_rows, 1), be.reshape(np_rows, 1))


def _combine_bias(part, b):
    np_rows = part.shape[1]

    def body(p_ref, b_ref, o_ref):
        o_ref[...] = p_ref[0] + p_ref[1] + b_ref[...]

    return pl.pallas_call(
        body,
        out_shape=jax.ShapeDtypeStruct((np_rows, BP), jnp.float32),
    )(part, b.reshape(np_rows, 1))


def _pad_edges(src, dst, w, cin, E, K):
    tot = NW * K * E
    pad = tot - src.shape[0]
    src = jnp.pad(src, (0, pad))
    dst = jnp.pad(dst, (0, pad))
    wf = jnp.pad(w.reshape(-1, cin).T, ((0, 0), (0, pad)))
    return src, dst, wf


def _pad_vec(v, n):
    return jnp.pad(v, (0, n - v.shape[0]))


def kernel(x, src1, dst1, w1, b1, g1, be1, src2, dst2, w2, b2, g2, be2,
           src3, dst3, w3, b3, g3, be3, src4, dst4, w4, b4, g4, be4,
           src5, dst5, w5, b5):
    n0 = x.shape[1] // 3

    xr = x.reshape(B, n0, 3).transpose(1, 2, 0)
    xr = jnp.pad(xr, ((0, 0), (0, 0), (0, BP - B))).reshape(n0, 3 * BP)

    def level(h, src, dst, w, cin, n_out_pad, E):
        nnz = src.shape[0]
        K = -(-nnz // (NW * E))
        s, d, wf = _pad_edges(src, dst, w, cin, E, K)
        kfn = _sc_pool(h.shape[0], cin, n_out_pad, E, K)
        return kfn(h, s, d, wf)

    p1 = level(xr, src1, dst1, w1, 3, 5120, 64)
    h1 = _bn_relu(p1, _pad_vec(g1, 5120), _pad_vec(be1, 5120))

    p2 = level(h1, src2, dst2, w2, 1, 1024, 128)
    h2 = _bn_relu(p2, _pad_vec(g2, 1024), _pad_vec(be2, 1024))

    p3 = level(h2, src3, dst3, w3, 1, 256, 128)
    h3 = _bn_relu(p3, _pad_vec(g3, 256), _pad_vec(be3, 256))

    p4 = level(h3, src4, dst4, w4, 1, 64, 32)
    h4 = _bn_relu(p4, _pad_vec(g4, 64), _pad_vec(be4, 64))

    p5 = level(h4, src5, dst5, w5, 1, 16, 16)
    y5 = _combine_bias(p5, b5)

    return y5[:, :B].T


# speedup vs baseline: 1.9387x; 1.9387x over previous
import functools

import jax
import jax.numpy as jnp
from jax import lax
from jax.experimental import pallas as pl
from jax.experimental.pallas import tpu as pltpu
from jax.experimental.pallas import tpu_sc as plsc

B = 200
LANES = 16
BP = 256
BW = 128
NVP = BW // LANES
NH = 2
NC, NS = 2, 16
NW = NC * NS


def _sc_pool(n_rows, cin, n_out_pad, E, K):
    RPT = max(8, n_out_pad // NS)
    NACT = n_out_pad // RPT
    Z = min(16, RPT)
    nz = RPT // Z
    SRCN = NH * cin * K * E
    mesh = plsc.VectorSubcoreMesh(core_axis_name="c", subcore_axis_name="s")

    scratch = [
        pltpu.VMEM((SRCN,), jnp.int32),
        pltpu.VMEM((cin * K * E,), jnp.float32),
        pltpu.VMEM((Z, BW), jnp.float32),
        pltpu.VMEM_SHARED((n_out_pad, BW), jnp.float32),
    ]
    for _ in range(2 * cin + 2):
        scratch.append(pltpu.VMEM((E, BW), jnp.float32))
    scratch.append(pltpu.VMEM((K * E,), jnp.int32))
    scratch += [pltpu.SemaphoreType.DMA] * 4

    @functools.partial(
        pl.kernel,
        mesh=mesh,
        compiler_params=pltpu.CompilerParams(use_tc_tiling_on_sc=False),
        out_type=jax.ShapeDtypeStruct((NC, NH, n_out_pad, BW), jnp.float32),
        scratch_types=scratch,
    )
    def kfn(h_hbm, src_hbm, dst_hbm, w_hbm, out_hbm, src_v, w_v,
            zbuf, acc, *bufs):
        rv = (bufs[0:cin], bufs[cin:2 * cin])
        ov = bufs[2 * cin:2 * cin + 2]
        dst_v = bufs[2 * cin + 2]
        sg = bufs[2 * cin + 3:2 * cin + 5]
        ss = bufs[2 * cin + 5:2 * cin + 7]
        cid = lax.axis_index("c")
        sid = lax.axis_index("s")
        wid = cid * NS + sid

        zv = jnp.zeros((LANES,), jnp.float32)
        for i in range(Z):
            for j in range(NVP):
                zbuf[i, pl.ds(j * LANES, LANES)] = zv

        pltpu.sync_copy(src_hbm.at[pl.ds(wid * SRCN, SRCN)], src_v)
        pltpu.sync_copy(w_hbm.at[pl.ds(wid * cin * K * E, cin * K * E)], w_v)
        pltpu.sync_copy(dst_hbm.at[pl.ds(wid * K * E, K * E)], dst_v)

        def zero_acc():
            @pl.when(sid < NACT)
            def _():
                for t in range(nz):
                    pltpu.sync_copy(zbuf, acc.at[pl.ds(sid * RPT + t * Z, Z)])

        def src_slice(h, c, k):
            return src_v.at[pl.ds(((h * cin + c) * K + k) * E, E)]

        def gather(h, k, b):
            for c in range(cin):
                pltpu.async_copy(h_hbm.at[src_slice(h, c, k)], rv[b][c],
                                 sg[b])

        def gather_wait(h, k, b):
            for c in range(cin):
                pltpu.make_async_copy(h_hbm.at[src_slice(h, c, k)], rv[b][c],
                                      sg[b]).wait()

        def dst_slice(k):
            return dst_v.at[pl.ds(k * E, E)]

        def compute(k, b):
            def group_body(gi, c2):
                wv = [w_v[pl.ds((c * K + k) * E + gi * LANES, LANES)]
                      for c in range(cin)]
                for l in range(LANES):
                    e = gi * LANES + l
                    for j in range(NVP):
                        sl = pl.ds(j * LANES, LANES)
                        r = rv[b][0][e, sl] * wv[0][l]
                        for c in range(1, cin):
                            r += rv[b][c][e, sl] * wv[c][l]
                        ov[b][e, sl] = r
                return c2

            lax.fori_loop(0, E // LANES, group_body, 0)

        @pl.loop(0, NH)
        def _(h):
            zero_acc()
            plsc.subcore_barrier()

            gather(h, 0, 0)
            if K > 1:
                gather(h, 1, 1)

            if K <= 2:
                for k in range(K):
                    b = k % 2
                    gather_wait(h, k, b)
                    compute(k, b)
                    pltpu.async_copy(ov[b], acc.at[dst_slice(k)], ss[b],
                                     add=True)
            else:
                @pl.loop(0, K, step=2)
                def _(kk):
                    for b in range(2):
                        k = kk + b
                        gather_wait(h, k, b)

                        @pl.when(kk >= 2)
                        def _():
                            pltpu.make_async_copy(
                                ov[b], acc.at[dst_slice(k - 2)], ss[b]).wait()

                        compute(k, b)
                        pltpu.async_copy(ov[b], acc.at[dst_slice(k)], ss[b],
                                         add=True)

                        @pl.when(k + 2 < K)
                        def _():
                            for c in range(cin):
                                pltpu.async_copy(
                                    h_hbm.at[src_slice(h, c, k + 2)],
                                    rv[b][c], sg[b])

            if K >= 2:
                pltpu.make_async_copy(ov[K % 2], acc.at[dst_slice(K - 2)],
                                      ss[K % 2]).wait()
            pltpu.make_async_copy(ov[(K - 1) % 2], acc.at[dst_slice(K - 1)],
                                  ss[(K - 1) % 2]).wait()
            plsc.subcore_barrier()

            @pl.when(sid < NACT)
            def _():
                pltpu.sync_copy(acc.at[pl.ds(sid * RPT, RPT)],
                                out_hbm.at[cid, h, pl.ds(sid * RPT, RPT)])

    return kfn


_TCOL = 512


def _fmt_x(x):
    m = x.shape[1]
    nb = -(-m // _TCOL)

    def body(x_ref, o_ref):
        t = x_ref[...].T
        o_ref[...] = jnp.stack([
            t[:, :BW],
            jnp.concatenate(
                [t[:, BW:], jnp.zeros((_TCOL, BP - B), jnp.float32)],
                axis=1),
        ])

    return pl.pallas_call(
        body,
        grid=(nb,),
        in_specs=[pl.BlockSpec((B, _TCOL), lambda k: (0, k))],
        out_specs=pl.BlockSpec((2, _TCOL, BW), lambda k: (0, k, 0)),
        out_shape=jax.ShapeDtypeStruct((2, nb * _TCOL, BW), jnp.float32),
    )(x)


def _bn_relu(part, g, be):
    np_rows = part.shape[2]

    def body(p_ref, g_ref, be_ref, o_ref):
        t = jnp.concatenate(
            [p_ref[0, h] + p_ref[1, h] for h in range(NH)], axis=1)
        lane = lax.broadcasted_iota(jnp.int32, (np_rows, BP), 1)
        mask = lane < B
        m = jnp.sum(t, axis=1, keepdims=True) * (1.0 / B)
        d = jnp.where(mask, t - m, 0.0)
        v = jnp.sum(d * d, axis=1, keepdims=True) * (1.0 / B)
        y = d * lax.rsqrt(v + 1e-5) * g_ref[...] + be_ref[...]
        y = jnp.where(mask, jnp.maximum(y, 0.0), 0.0)
        o_ref[...] = jnp.stack([y[:, h * BW:(h + 1) * BW] for h in range(NH)])

    out = pl.pallas_call(
        body,
        out_shape=jax.ShapeDtypeStruct((NH, np_rows, BW), jnp.float32),
    )(part, g.reshape(np_rows, 1), be.reshape(np_rows, 1))
    return out.reshape(NH * np_rows, BW)


def _combine_bias(part, b):
    np_rows = part.shape[2]

    def body(p_ref, b_ref, o_ref):
        t = jnp.concatenate(
            [p_ref[0, h] + p_ref[1, h] for h in range(NH)], axis=1)
        o_ref[...] = t + b_ref[...]

    return pl.pallas_call(
        body,
        out_shape=jax.ShapeDtypeStruct((np_rows, BP), jnp.float32),
    )(part, b.reshape(np_rows, 1))


def _round16(n):
    return -(-n // 16) * 16


def _pad_vec(v, n):
    return jnp.pad(v, (0, n - v.shape[0]))


def kernel(x, src1, dst1, w1, b1, g1, be1, src2, dst2, w2, b2, g2, be2,
           src3, dst3, w3, b3, g3, be3, src4, dst4, w4, b4, g4, be4,
           src5, dst5, w5, b5):
    xr = _fmt_x(x)
    n_rows1 = xr.shape[1]

    def level(h, n_rows, srcs, dst, w, n_out_pad, emax):
        cin = len(srcs)
        nnz = dst.shape[0]
        E = max(32, min(emax, _round16(-(-nnz // NW))))
        K = -(-nnz // (NW * E))
        if K > 1:
            K = -(-K // 2) * 2
        tot = NW * K * E
        pad = tot - nnz
        s = jnp.stack([jnp.pad(sc, (0, pad)) for sc in srcs])
        dpad = jnp.arange(pad, dtype=jnp.int32) % n_out_pad
        offs = jnp.arange(NH, dtype=jnp.int32) * n_rows
        s = s[None, :, :] + offs[:, None, None]
        s = s.reshape(NH, cin, NW, K * E).transpose(2, 0, 1, 3).reshape(-1)
        d = jnp.concatenate([dst, dpad]).reshape(-1)
        wf = jnp.pad(w.reshape(-1, cin), ((0, pad), (0, 0)))
        wf = wf.reshape(NW, K * E, cin).transpose(0, 2, 1).reshape(-1)
        kfn = _sc_pool(n_rows, cin, n_out_pad, E, K)
        return kfn(h, s, d, wf)

    p1 = level(xr.reshape(NH * n_rows1, BW), n_rows1,
               [src1 * 3, src1 * 3 + 1, src1 * 3 + 2], dst1, w1, 5120, 64)
    h1 = _bn_relu(p1, _pad_vec(g1, 5120), _pad_vec(be1, 5120))

    p2 = level(h1, 5120, [src2], dst2, w2, 1024, 80)
    h2 = _bn_relu(p2, _pad_vec(g2, 1024), _pad_vec(be2, 1024))

    p3 = level(h2, 1024, [src3], dst3, w3, 256, 64)
    h3 = _bn_relu(p3, _pad_vec(g3, 256), _pad_vec(be3, 256))

    p4 = level(h3, 256, [src4], dst4, w4, 64, 96)
    h4 = _bn_relu(p4, _pad_vec(g4, 64), _pad_vec(be4, 64))

    p5 = level(h4, 64, [src5], dst5, w5, 16, 96)
    y5 = _combine_bias(p5, b5)

    return y5[:, :B].T
